# Initial kernel scaffold; baseline (speedup 1.0000x reference)
import jax
import jax.numpy as jnp
from jax.experimental import pallas as pl


def kernel(x, edge_index, edge_attn, W1, b1, W2, b2):
    raise NotImplementedError("write your pallas kernel here")


# speedup vs baseline: 4.0354x; 4.0354x over previous
import functools

import jax
import jax.numpy as jnp
from jax import lax
from jax.experimental import pallas as pl
from jax.experimental.pallas import tpu as pltpu
from jax.experimental.pallas import tpu_sc as plsc

N = 10000
E = 320000
D = 128

NC = 2
NS = 16
B = 80
EDGES_PER_TILE = E // (NC * NS)
CHUNKS = EDGES_PER_TILE // B
ROWS_PER_TILE = 624
TAIL_ROWS = N - NS * ROWS_PER_TILE
ZR = 208


def _sc_body(x_hbm, src_hbm, dst_hbm, attn_hbm, hp_hbm,
             acc, srcb, dstb, attnb, rows, zbuf, sem):
    c = lax.axis_index("c")
    s = lax.axis_index("s")

    def zrow(i, carry):
        for j in range(D // 16):
            zbuf[i, pl.ds(j * 16, 16)] = jnp.zeros((16,), jnp.float32)
        return carry

    lax.fori_loop(0, ZR, zrow, 0)
    for q in range(ROWS_PER_TILE // ZR):
        pltpu.sync_copy(zbuf, acc.at[pl.ds(s * ROWS_PER_TILE + q * ZR, ZR)])

    @pl.when(s == NS - 1)
    def _zero_tail():
        pltpu.sync_copy(zbuf.at[pl.ds(0, TAIL_ROWS)],
                        acc.at[pl.ds(NS * ROWS_PER_TILE, TAIL_ROWS)])

    plsc.subcore_barrier()

    base0 = (c * NS + s) * EDGES_PER_TILE

    def chunk(k, carry):
        base = base0 + k * B
        pltpu.sync_copy(src_hbm.at[pl.ds(base, B)], srcb.at[0])
        pltpu.sync_copy(dst_hbm.at[pl.ds(base, B)], dstb.at[0])
        pltpu.sync_copy(attn_hbm.at[pl.ds(base, B)], attnb)
        pltpu.async_copy(x_hbm.at[srcb.at[0]], rows, sem).wait()

        def rowscale(g, rcarry):
            av = attnb[pl.ds(g * 16, 16)]
            for t in range(16):
                i = g * 16 + t
                a = jnp.full((16,), av[t], jnp.float32)
                for j in range(D // 16):
                    rows[i, pl.ds(j * 16, 16)] = (
                        rows[i, pl.ds(j * 16, 16)] * a)
            return rcarry

        lax.fori_loop(0, B // 16, rowscale, 0)
        pltpu.sync_copy(rows, acc.at[dstb.at[0]], add=True)
        return carry

    lax.fori_loop(0, CHUNKS, chunk, 0)
    plsc.subcore_barrier()

    pltpu.sync_copy(acc.at[pl.ds(s * ROWS_PER_TILE, ROWS_PER_TILE)],
                    hp_hbm.at[c, pl.ds(s * ROWS_PER_TILE, ROWS_PER_TILE)])

    @pl.when(s == NS - 1)
    def _drain_tail():
        pltpu.sync_copy(acc.at[pl.ds(NS * ROWS_PER_TILE, TAIL_ROWS)],
                        hp_hbm.at[c, pl.ds(NS * ROWS_PER_TILE, TAIL_ROWS)])


def _sc_message_passing(x, src, dst, attn):
    mesh = plsc.VectorSubcoreMesh(core_axis_name="c", subcore_axis_name="s")
    kern = pl.kernel(
        _sc_body,
        mesh=mesh,
        out_type=jax.ShapeDtypeStruct((NC, N, D), jnp.float32),
        scratch_types=[
            pltpu.VMEM_SHARED((N, D), jnp.float32),
            pltpu.VMEM((1, B), jnp.int32),
            pltpu.VMEM((1, B), jnp.int32),
            pltpu.VMEM((B,), jnp.float32),
            pltpu.VMEM((B, D), jnp.float32),
            pltpu.VMEM((ZR, D), jnp.float32),
            pltpu.SemaphoreType.DMA,
        ],
    )
    return kern(x, src, dst, attn)


def _tc_body(x_ref, h0_ref, h1_ref, w1_ref, b1_ref, w2_ref, b2_ref, o_ref):
    x = x_ref[...]
    hn = h0_ref[...] + h1_ref[...]
    u = x + hn
    v = x * hn
    dn = (((1,), (1,)), ((), ()))
    y1 = lax.dot_general(u, w1_ref[...], dn,
                         preferred_element_type=jnp.float32) + b1_ref[...]
    y1 = jnp.where(y1 >= 0, y1, y1 * 0.01)
    y2 = lax.dot_general(v, w2_ref[...], dn,
                         preferred_element_type=jnp.float32) + b2_ref[...]
    y2 = jnp.where(y2 >= 0, y2, y2 * 0.01)
    o_ref[...] = y1 + y2


def _tc_dense(x, h0, h1, W1, b1, W2, b2):
    BN = 1000
    grid = (N // BN,)
    row_spec = pl.BlockSpec((BN, D), lambda i: (i, 0))
    full_spec = pl.BlockSpec((D, D), lambda i: (0, 0))
    bias_spec = pl.BlockSpec((1, D), lambda i: (0, 0))
    return pl.pallas_call(
        _tc_body,
        grid=grid,
        in_specs=[row_spec, row_spec, row_spec, full_spec, bias_spec,
                  full_spec, bias_spec],
        out_specs=row_spec,
        out_shape=jax.ShapeDtypeStruct((N, D), jnp.float32),
    )(x, h0, h1, W1, b1, W2, b2)


@jax.jit
def kernel(x, edge_index, edge_attn, W1, b1, W2, b2):
    src = edge_index[0]
    dst = edge_index[1]
    attn = edge_attn.reshape(E)
    hp = _sc_message_passing(x, src, dst, attn)
    out = _tc_dense(x, hp[0], hp[1], W1, b1.reshape(1, D), W2,
                    b2.reshape(1, D))
    return out


# speedup vs baseline: 7.8540x; 1.9463x over previous
import functools

import jax
import jax.numpy as jnp
from jax import lax
from jax.experimental import pallas as pl
from jax.experimental.pallas import tpu as pltpu
from jax.experimental.pallas import tpu_sc as plsc

N = 10000
E = 320000
D = 128

NC = 2
NS = 16
B = 80
EDGES_PER_TILE = E // (NC * NS)
CHUNKS = EDGES_PER_TILE // B
ROWS_PER_TILE = 624
TAIL_ROWS = N - NS * ROWS_PER_TILE
ZR = 208


def _sc_body(x_hbm, src_hbm, dst_hbm, attn_hbm, hp_hbm,
             acc, srcb, dstb, attnb, rows, zbuf,
             sem_i0, sem_i1, sem_g0, sem_g1, sem_s0, sem_s1):
    sem_i = (sem_i0, sem_i1)
    sem_g = (sem_g0, sem_g1)
    sem_s = (sem_s0, sem_s1)
    c = lax.axis_index("c")
    s = lax.axis_index("s")

    def zrow(i, carry):
        for j in range(D // 16):
            zbuf[i, pl.ds(j * 16, 16)] = jnp.zeros((16,), jnp.float32)
        return carry

    lax.fori_loop(0, ZR, zrow, 0)
    for q in range(ROWS_PER_TILE // ZR):
        pltpu.sync_copy(zbuf, acc.at[pl.ds(s * ROWS_PER_TILE + q * ZR, ZR)])

    @pl.when(s == NS - 1)
    def _zero_tail():
        pltpu.sync_copy(zbuf.at[pl.ds(0, TAIL_ROWS)],
                        acc.at[pl.ds(NS * ROWS_PER_TILE, TAIL_ROWS)])

    plsc.subcore_barrier()

    base0 = (c * NS + s) * EDGES_PER_TILE

    def issue_idx(ki, b):
        base = base0 + ki * B
        pltpu.async_copy(src_hbm.at[pl.ds(base, B)], srcb.at[b], sem_i[b])
        pltpu.async_copy(dst_hbm.at[pl.ds(base, B)], dstb.at[b], sem_i[b])
        pltpu.async_copy(attn_hbm.at[pl.ds(base, B)], attnb.at[b], sem_i[b])

    def wait_idx(ki, b):
        base = base0 + ki * B
        pltpu.make_async_copy(src_hbm.at[pl.ds(base, B)], srcb.at[b],
                              sem_i[b]).wait()
        pltpu.make_async_copy(dst_hbm.at[pl.ds(base, B)], dstb.at[b],
                              sem_i[b]).wait()
        pltpu.make_async_copy(attn_hbm.at[pl.ds(base, B)], attnb.at[b],
                              sem_i[b]).wait()

    def issue_gather(b):
        pltpu.async_copy(x_hbm.at[srcb.at[b]], rows.at[b], sem_g[b])

    def wait_gather(b):
        pltpu.make_async_copy(x_hbm.at[srcb.at[b]], rows.at[b],
                              sem_g[b]).wait()

    def issue_scatter(b):
        pltpu.async_copy(rows.at[b], acc.at[dstb.at[b]], sem_s[b], add=True)

    def wait_scatter(b):
        pltpu.make_async_copy(rows.at[b], acc.at[dstb.at[b]],
                              sem_s[b]).wait()

    def scale(b):
        def rowscale(g, rcarry):
            av = attnb[b, pl.ds(g * 16, 16)]
            for t in range(16):
                a = jnp.full((16,), av[t], jnp.float32)
                for j in range(D // 16):
                    rows[b, g * 16 + t, pl.ds(j * 16, 16)] = (
                        rows[b, g * 16 + t, pl.ds(j * 16, 16)] * a)
            return rcarry

        lax.fori_loop(0, B // 16, rowscale, 0)

    issue_idx(0, 0)
    wait_idx(0, 0)
    issue_gather(0)
    wait_gather(0)
    scale(0)
    issue_scatter(0)
    issue_idx(1, 1)
    wait_idx(1, 1)
    issue_gather(1)

    def loop_body(k, carry):
        for off in range(2):
            ki = k + off
            b = 1 - off
            nb = off
            knext = jnp.minimum(ki + 1, CHUNKS - 1)
            wait_scatter(nb)
            issue_idx(knext, nb)
            wait_gather(b)
            wait_idx(knext, nb)
            issue_gather(nb)
            scale(b)
            issue_scatter(b)
        return carry

    lax.fori_loop(0, (CHUNKS - 1) // 2, lambda i, cy: loop_body(1 + 2 * i, cy),
                  0)
    wait_gather(1)
    wait_scatter(0)
    plsc.subcore_barrier()

    pltpu.sync_copy(acc.at[pl.ds(s * ROWS_PER_TILE, ROWS_PER_TILE)],
                    hp_hbm.at[c, pl.ds(s * ROWS_PER_TILE, ROWS_PER_TILE)])

    @pl.when(s == NS - 1)
    def _drain_tail():
        pltpu.sync_copy(acc.at[pl.ds(NS * ROWS_PER_TILE, TAIL_ROWS)],
                        hp_hbm.at[c, pl.ds(NS * ROWS_PER_TILE, TAIL_ROWS)])


def _sc_message_passing(x, src, dst, attn):
    mesh = plsc.VectorSubcoreMesh(core_axis_name="c", subcore_axis_name="s")
    kern = pl.kernel(
        _sc_body,
        mesh=mesh,
        out_type=jax.ShapeDtypeStruct((NC, N, D), jnp.float32),
        scratch_types=[
            pltpu.VMEM_SHARED((N, D), jnp.float32),
            pltpu.VMEM((2, B), jnp.int32),
            pltpu.VMEM((2, B), jnp.int32),
            pltpu.VMEM((2, B), jnp.float32),
            pltpu.VMEM((2, B, D), jnp.float32),
            pltpu.VMEM((ZR, D), jnp.float32),
            pltpu.SemaphoreType.DMA,
            pltpu.SemaphoreType.DMA,
            pltpu.SemaphoreType.DMA,
            pltpu.SemaphoreType.DMA,
            pltpu.SemaphoreType.DMA,
            pltpu.SemaphoreType.DMA,
        ],
    )
    return kern(x, src, dst, attn)


def _tc_body(x_ref, h0_ref, h1_ref, w1_ref, b1_ref, w2_ref, b2_ref, o_ref):
    x = x_ref[...]
    hn = h0_ref[...] + h1_ref[...]
    u = x + hn
    v = x * hn
    dn = (((1,), (1,)), ((), ()))
    y1 = lax.dot_general(u, w1_ref[...], dn,
                         preferred_element_type=jnp.float32) + b1_ref[...]
    y1 = jnp.where(y1 >= 0, y1, y1 * 0.01)
    y2 = lax.dot_general(v, w2_ref[...], dn,
                         preferred_element_type=jnp.float32) + b2_ref[...]
    y2 = jnp.where(y2 >= 0, y2, y2 * 0.01)
    o_ref[...] = y1 + y2


def _tc_dense(x, h0, h1, W1, b1, W2, b2):
    BN = 1000
    grid = (N // BN,)
    row_spec = pl.BlockSpec((BN, D), lambda i: (i, 0))
    full_spec = pl.BlockSpec((D, D), lambda i: (0, 0))
    bias_spec = pl.BlockSpec((1, D), lambda i: (0, 0))
    return pl.pallas_call(
        _tc_body,
        grid=grid,
        in_specs=[row_spec, row_spec, row_spec, full_spec, bias_spec,
                  full_spec, bias_spec],
        out_specs=row_spec,
        out_shape=jax.ShapeDtypeStruct((N, D), jnp.float32),
    )(x, h0, h1, W1, b1, W2, b2)


@jax.jit
def kernel(x, edge_index, edge_attn, W1, b1, W2, b2):
    src = edge_index[0]
    dst = edge_index[1]
    attn = edge_attn.reshape(E)
    hp = _sc_message_passing(x, src, dst, attn)
    out = _tc_dense(x, hp[0], hp[1], W1, b1.reshape(1, D), W2,
                    b2.reshape(1, D))
    return out
